# baseline (speedup 1.0000x reference)
import jax
import jax.numpy as jnp
from jax import lax
from jax.experimental import pallas as pl
from jax.experimental.pallas import tpu as pltpu
from jax.experimental.pallas import tpu_sc as plsc

N = 10000
E = 320000
D = 128
G = 3
NC = 2
NS = 16
NW = NC * NS
K = 128
ECP = 10240
NCHUNK = ECP // K
NPAD = 10240
RPT = NPAD // NS
DEG_LEN = G * NPAD
DPT = DEG_LEN // NS
R = 1000


def _prep_edges(edge_index, edge_weight, goff):
    pad = NW * ECP - E
    zi = jnp.zeros((pad,), jnp.int32)
    srcoff = (jnp.concatenate([edge_index[0], zi]) + goff).reshape(
        NW, NCHUNK, K)
    dstf = jnp.concatenate([edge_index[1], zi])
    dst = dstf.reshape(NW, NCHUNK, K)
    dstoff = (dstf + goff).reshape(NW, NCHUNK, K)
    ew = jnp.concatenate([edge_weight, jnp.zeros((pad,), jnp.float32)])
    return srcoff, dst, dstoff, ew.reshape(NW, NCHUNK, K)


def _deg_body(dst_hbm, ew_hbm, out_hbm, dstv, ewv, zb, acc, sem):
    cid = lax.axis_index("c")
    sid = lax.axis_index("s")
    wid = sid * NC + cid
    zero = jnp.zeros((16,), jnp.float32)

    def zloop(i, carry):
        zb[pl.ds(pl.multiple_of(i * 16, 16), 16)] = zero
        return carry

    lax.fori_loop(0, DPT // 16, zloop, 0)
    pltpu.sync_copy(zb, acc.at[pl.ds(sid * DPT, DPT)])
    pltpu.sync_copy(dst_hbm.at[wid], dstv)
    pltpu.sync_copy(ew_hbm.at[wid], ewv)
    plsc.subcore_barrier()

    def floop(s, carry):
        for q in range(8):
            c = s * 8 + q
            pltpu.async_copy(ewv.at[c], acc.at[dstv.at[c]], sem, add=True)
        for q in range(8):
            c = s * 8 + q
            pltpu.make_async_copy(ewv.at[c], acc.at[dstv.at[c]], sem).wait()
        return carry

    lax.fori_loop(0, (G * NCHUNK) // 8, floop, 0)
    plsc.subcore_barrier()
    pltpu.sync_copy(acc.at[pl.ds(sid * DPT, DPT)],
                    out_hbm.at[cid, pl.ds(sid * DPT, DPT)])


_SC_CACHE = {}


def _deg_call(*args):
    if "deg" not in _SC_CACHE:
        _SC_CACHE["deg"] = pl.kernel(
            _deg_body,
            out_type=jax.ShapeDtypeStruct((NC, DEG_LEN), jnp.float32),
            mesh=plsc.VectorSubcoreMesh(core_axis_name="c",
                                        subcore_axis_name="s",
                                        num_cores=NC, num_subcores=NS),
            scratch_types=[
                pltpu.VMEM((G * NCHUNK, K), jnp.int32),
                pltpu.VMEM((G * NCHUNK, K), jnp.float32),
                pltpu.VMEM((DPT,), jnp.float32),
                pltpu.VMEM_SHARED((DEG_LEN,), jnp.float32),
                pltpu.SemaphoreType.DMA,
            ],
        )
    return _SC_CACHE["deg"](*args)


def _lane_bcast(vec, jj):
    idx = jnp.full((16, 1), jj, jnp.int32)
    return lax.gather(
        vec, idx,
        lax.GatherDimensionNumbers(offset_dims=(), collapsed_slice_dims=(0,),
                                   start_index_map=(0,)),
        (1,), mode=lax.GatherScatterMode.PROMISE_IN_BOUNDS)


def _seg_body(y_hbm, src_hbm, dst_hbm, ew_hbm, out_hbm,
              sbuf, dbuf, ebuf, rbuf, acc, isem0, isem1, gsem0, gsem1):
    cid = lax.axis_index("c")
    sid = lax.axis_index("s")
    wid = sid * NC + cid
    zero = jnp.zeros((16,), jnp.float32)
    isem = (isem0, isem1)
    gsem = (gsem0, gsem1)
    base = sid * RPT

    def start_idx(r, b):
        pltpu.async_copy(src_hbm.at[wid, r], sbuf.at[b], isem[b])
        pltpu.async_copy(dst_hbm.at[wid, r], dbuf.at[b], isem[b])
        pltpu.async_copy(ew_hbm.at[wid, r], ebuf.at[b], isem[b])

    def wait_idx(r, b):
        pltpu.make_async_copy(src_hbm.at[wid, r], sbuf.at[b], isem[b]).wait()
        pltpu.make_async_copy(dst_hbm.at[wid, r], dbuf.at[b], isem[b]).wait()
        pltpu.make_async_copy(ew_hbm.at[wid, r], ebuf.at[b], isem[b]).wait()

    for g in range(G):
        row0 = g * NCHUNK

        def zrow(j, carry):
            for t in range(8):
                rbuf[0, j, pl.ds(16 * t, 16)] = zero
            return carry

        lax.fori_loop(0, K, zrow, 0)
        for b in range(RPT // K):
            pltpu.sync_copy(rbuf.at[0], acc.at[pl.ds(base + b * K, K)])
        plsc.subcore_barrier()

        start_idx(row0 + 0, 0)
        start_idx(row0 + 1, 1)
        wait_idx(row0 + 0, 0)
        pltpu.async_copy(y_hbm.at[sbuf.at[0]], rbuf.at[0], gsem0)

        def pair(c2, carry):
            for b in range(2):
                c = 2 * c2 + b
                nb = 1 - b

                @pl.when(c < NCHUNK - 1)
                def _launch_next_gather():
                    wait_idx(row0 + c + 1, nb)
                    pltpu.async_copy(y_hbm.at[sbuf.at[nb]], rbuf.at[nb],
                                     gsem[nb])

                pltpu.make_async_copy(y_hbm.at[sbuf.at[b]], rbuf.at[b],
                                      gsem[b]).wait()

                def sgrp(q2, icarry):
                    ew16 = ebuf[b, pl.ds(q2 * 16, 16)]
                    for jj in range(16):
                        j = q2 * 16 + jj
                        w = _lane_bcast(ew16, jj)
                        for t in range(8):
                            sl = pl.ds(16 * t, 16)
                            rbuf[b, j, sl] = rbuf[b, j, sl] * w
                    return icarry

                lax.fori_loop(0, K // 16, sgrp, 0)
                pltpu.sync_copy(rbuf.at[b], acc.at[dbuf.at[b]], add=True)

                @pl.when(c < NCHUNK - 2)
                def _start_next_idx():
                    start_idx(row0 + c + 2, b)

            return carry

        lax.fori_loop(0, NCHUNK // 2, pair, 0)
        plsc.subcore_barrier()
        for b in range(RPT // K):
            pltpu.sync_copy(acc.at[pl.ds(base + b * K, K)],
                            out_hbm.at[cid, g, pl.ds(base + b * K, K)])
        plsc.subcore_barrier()


def _seg_call(*args):
    if "seg" not in _SC_CACHE:
        _SC_CACHE["seg"] = pl.kernel(
            _seg_body,
            out_type=jax.ShapeDtypeStruct((NC, G, NPAD, D), jnp.float32),
            mesh=plsc.VectorSubcoreMesh(core_axis_name="c",
                                        subcore_axis_name="s",
                                        num_cores=NC, num_subcores=NS),
            scratch_types=[
                pltpu.VMEM((2, K), jnp.int32),
                pltpu.VMEM((2, K), jnp.int32),
                pltpu.VMEM((2, K), jnp.float32),
                pltpu.VMEM((2, K, D), jnp.float32),
                pltpu.VMEM_SHARED((NPAD, D), jnp.float32),
                pltpu.SemaphoreType.DMA,
                pltpu.SemaphoreType.DMA,
                pltpu.SemaphoreType.DMA,
                pltpu.SemaphoreType.DMA,
            ],
        )
    return _SC_CACHE["seg"](*args)


def _full(shape):
    return pl.BlockSpec(shape, lambda *a: tuple(0 for _ in shape))


def _dinv_body(d_ref, o_ref):
    o_ref[...] = lax.rsqrt(d_ref[0] + d_ref[1] + 1.0)


def _finish_deg(degacc):
    return pl.pallas_call(
        _dinv_body,
        out_shape=jax.ShapeDtypeStruct((G, NPAD), jnp.float32),
    )(degacc.reshape(NC, G, NPAD))


def _fc_body(x_ref, W1, b1, W2, b2, W3, b3, o_ref):
    h = jnp.dot(x_ref[...], W1[...], preferred_element_type=jnp.float32)
    h = jnp.maximum(h + b1[...], 0.0)
    h = jnp.dot(h, W2[...], preferred_element_type=jnp.float32)
    h = jnp.maximum(h + b2[...], 0.0)
    h = jnp.dot(h, W3[...], preferred_element_type=jnp.float32)
    o_ref[...] = jnp.maximum(h + b3[...], 0.0)


def _mlp(x, W1, b1, W2, b2, W3, b3):
    return pl.pallas_call(
        _fc_body,
        grid=(N // R,),
        in_specs=[pl.BlockSpec((R, D), lambda i: (i, 0)),
                  _full((D, 256)), _full((1, 256)),
                  _full((256, D)), _full((1, D)),
                  _full((D, 64)), _full((1, 64))],
        out_specs=pl.BlockSpec((R, 64), lambda i: (i, 0)),
        out_shape=jax.ShapeDtypeStruct((N, 64), jnp.float32),
    )(x, W1, b1.reshape(1, -1), W2, b2.reshape(1, -1), W3, b3.reshape(1, -1))


def _pre_body(x_ref, W_ref, dv_ref, y_ref):
    y_ref[0] = jnp.dot(x_ref[...], W_ref[0],
                       preferred_element_type=jnp.float32) * dv_ref[0]


def _pre3(x, W, dv):
    return pl.pallas_call(
        _pre_body,
        grid=(G, N // R),
        in_specs=[pl.BlockSpec((R, D), lambda g, i: (i, 0)),
                  pl.BlockSpec((1, D, D), lambda g, i: (g, 0, 0)),
                  pl.BlockSpec((1, R, 1), lambda g, i: (g, i, 0))],
        out_specs=pl.BlockSpec((1, R, D), lambda g, i: (g, i, 0)),
        out_shape=jax.ShapeDtypeStruct((G, NPAD, D), jnp.float32),
    )(x, W, dv)


def _mid_body(acc_ref, y_ref, dv_ref, b_ref, W_ref, o_ref):
    a = acc_ref[0, 0] + acc_ref[1, 0] + y_ref[0]
    h = jnp.maximum(a * dv_ref[0] + b_ref[0], 0.0)
    o_ref[0] = jnp.dot(h, W_ref[0],
                       preferred_element_type=jnp.float32) * dv_ref[0]


def _mid3(acc, y, dv, b, W):
    return pl.pallas_call(
        _mid_body,
        grid=(G, N // R),
        in_specs=[pl.BlockSpec((NC, 1, R, D), lambda g, i: (0, g, i, 0)),
                  pl.BlockSpec((1, R, D), lambda g, i: (g, i, 0)),
                  pl.BlockSpec((1, R, 1), lambda g, i: (g, i, 0)),
                  pl.BlockSpec((1, 1, D), lambda g, i: (g, 0, 0)),
                  pl.BlockSpec((1, D, D), lambda g, i: (g, 0, 0))],
        out_specs=pl.BlockSpec((1, R, D), lambda g, i: (g, i, 0)),
        out_shape=jax.ShapeDtypeStruct((G, NPAD, D), jnp.float32),
    )(acc, y, dv, b, W)


def _post_body(acc_ref, y_ref, dv_ref, b_ref, W1, b1, W2, b2, W3, b3, o_ref):
    a = acc_ref[0, 0] + acc_ref[1, 0] + y_ref[0]
    h = jnp.maximum(a * dv_ref[0] + b_ref[0], 0.0)
    h = jnp.dot(h, W1[...], preferred_element_type=jnp.float32)
    h = jnp.maximum(h + b1[...], 0.0)
    h = jnp.dot(h, W2[...], preferred_element_type=jnp.float32)
    h = jnp.maximum(h + b2[...], 0.0)
    h = jnp.dot(h, W3[...], preferred_element_type=jnp.float32)
    o_ref[0] = jnp.maximum(h + b3[...], 0.0)


def _post3(acc, y, dv, b, W1, b1, W2, b2, W3, b3):
    return pl.pallas_call(
        _post_body,
        grid=(G, N // R),
        in_specs=[pl.BlockSpec((NC, 1, R, D), lambda g, i: (0, g, i, 0)),
                  pl.BlockSpec((1, R, D), lambda g, i: (g, i, 0)),
                  pl.BlockSpec((1, R, 1), lambda g, i: (g, i, 0)),
                  pl.BlockSpec((1, 1, D), lambda g, i: (g, 0, 0)),
                  _full((D, 256)), _full((1, 256)),
                  _full((256, D)), _full((1, D)),
                  _full((D, 64)), _full((1, 64))],
        out_specs=pl.BlockSpec((1, R, 64), lambda g, i: (g, i, 0)),
        out_shape=jax.ShapeDtypeStruct((G, N, 64), jnp.float32),
    )(acc, y, dv, b, W1, b1.reshape(1, -1),
      W2, b2.reshape(1, -1), W3, b3.reshape(1, -1))


def _addf_body(x0_ref, o_ref, out_ref):
    out_ref[...] = x0_ref[...] + o_ref[0] + o_ref[1] + o_ref[2]


def _addf(x0, o):
    return pl.pallas_call(
        _addf_body,
        grid=(N // R,),
        in_specs=[pl.BlockSpec((R, 64), lambda i: (i, 0)),
                  pl.BlockSpec((G, R, 64), lambda i: (0, i, 0))],
        out_specs=pl.BlockSpec((R, 64), lambda i: (i, 0)),
        out_shape=jax.ShapeDtypeStruct((N, 64), jnp.float32),
    )(x0, o)


def kernel(X, edge_index1, edge_weight1, edge_index2, edge_weight2,
           edge_index3, edge_weight3,
           gcn1_W1, gcn1_b1, gcn2_W1, gcn2_b1,
           gcn1_W2, gcn1_b2, gcn2_W2, gcn2_b2,
           gcn1_W3, gcn1_b3, gcn2_W3, gcn2_b3,
           lin1_W1, lin1_b1, lin1_W2, lin1_b2, lin1_W3, lin1_b3,
           lin2_W1, lin2_b1, lin2_W2, lin2_b2, lin2_W3, lin2_b3):
    srcs, dsts, doffs, ews = [], [], [], []
    for g, (ei, ew) in enumerate([(edge_index1, edge_weight1),
                                  (edge_index2, edge_weight2),
                                  (edge_index3, edge_weight3)]):
        srcoff, dst, dstoff, eww = _prep_edges(ei, ew, g * NPAD)
        srcs.append(srcoff)
        dsts.append(dst)
        doffs.append(dstoff)
        ews.append(eww)
    src_all = jnp.concatenate(srcs, axis=1)
    dst_all = jnp.concatenate(dsts, axis=1)
    doff_all = jnp.concatenate(doffs, axis=1)
    ew_all = jnp.concatenate(ews, axis=1)

    degacc = _deg_call(doff_all, ew_all)
    dinv_all = _finish_deg(degacc)
    dv = dinv_all[:, :N].reshape(G, N, 1)

    X0 = _mlp(X, lin1_W1, lin1_b1, lin1_W2, lin1_b2, lin1_W3, lin1_b3)

    Wa = jnp.stack([gcn1_W1, gcn1_W2, gcn1_W3])
    ba = jnp.stack([gcn1_b1, gcn1_b2, gcn1_b3]).reshape(G, 1, D)
    Wb = jnp.stack([gcn2_W1, gcn2_W2, gcn2_W3])
    bb = jnp.stack([gcn2_b1, gcn2_b2, gcn2_b3]).reshape(G, 1, D)

    y1 = _pre3(X, Wa, dv)
    acc1 = _seg_call(y1.reshape(G * NPAD, D), src_all, dst_all, ew_all)
    y2 = _mid3(acc1, y1, dv, ba, Wb)
    acc2 = _seg_call(y2.reshape(G * NPAD, D), src_all, dst_all, ew_all)
    o = _post3(acc2, y2, dv, bb,
               lin2_W1, lin2_b1, lin2_W2, lin2_b2, lin2_W3, lin2_b3)

    Xout = _addf(X0, o)
    return (Xout, o[0], o[1], o[2])


# speedup vs baseline: 1.0388x; 1.0388x over previous
import jax
import jax.numpy as jnp
from jax import lax
from jax.experimental import pallas as pl
from jax.experimental.pallas import tpu as pltpu
from jax.experimental.pallas import tpu_sc as plsc

N = 10000
E = 320000
D = 128
G = 3
NC = 2
NS = 16
NW = NC * NS
K = 128
NCK0 = 120
NCK1 = 40
NCKM = NCK0
NPAD = 10240
RPT = NPAD // NS
DEG_LEN = G * NPAD
DPT = DEG_LEN // NS
R = 1000


def _prep_edges(edge_index, edge_weight, goff):
    pad = NS * (NCK0 + NCK1) * K - E

    def split(flat, dtype):
        a = flat.reshape(NS * (NCK0 + NCK1), K)
        a0 = a[:NS * NCK0].reshape(NS, NCK0, K)
        a1 = a[NS * NCK0:].reshape(NS, NCK1, K)
        a1 = jnp.concatenate(
            [a1, jnp.zeros((NS, NCK0 - NCK1, K), dtype)], axis=1)
        return jnp.concatenate([a0, a1], axis=0)

    zi = jnp.zeros((pad,), jnp.int32)
    srcoff = split(jnp.concatenate([edge_index[0], zi]) + goff, jnp.int32)
    dstf = jnp.concatenate([edge_index[1], zi])
    dst = split(dstf, jnp.int32)
    dstoff = split(dstf + goff, jnp.int32)
    ew = split(jnp.concatenate(
        [edge_weight, jnp.zeros((pad,), jnp.float32)]), jnp.float32)
    return srcoff, dst, dstoff, ew


def _deg_body(dst_hbm, ew_hbm, out_hbm, dstv, ewv, zb, acc, sem):
    cid = lax.axis_index("c")
    sid = lax.axis_index("s")
    wid = cid * NS + sid
    zero = jnp.zeros((16,), jnp.float32)

    def zloop(i, carry):
        zb[pl.ds(pl.multiple_of(i * 16, 16), 16)] = zero
        return carry

    lax.fori_loop(0, DPT // 16, zloop, 0)
    pltpu.sync_copy(zb, acc.at[pl.ds(sid * DPT, DPT)])

    def scat8(r0, s):
        for q in range(8):
            c = r0 + s * 8 + q
            pltpu.async_copy(ewv.at[c], acc.at[dstv.at[c]], sem, add=True)
        for q in range(8):
            c = r0 + s * 8 + q
            pltpu.make_async_copy(ewv.at[c], acc.at[dstv.at[c]], sem).wait()

    @pl.when(cid == 0)
    def _core0():
        pltpu.sync_copy(dst_hbm.at[wid], dstv)
        pltpu.sync_copy(ew_hbm.at[wid], ewv)
        plsc.subcore_barrier()

        def floop(s, carry):
            scat8(0, s)
            return carry

        lax.fori_loop(0, (G * NCK0) // 8, floop, 0)

    @pl.when(cid == 1)
    def _core1():
        for g in range(G):
            sl = pl.ds(g * NCKM, NCK1)
            pltpu.sync_copy(dst_hbm.at[wid, sl], dstv.at[sl])
            pltpu.sync_copy(ew_hbm.at[wid, sl], ewv.at[sl])
        plsc.subcore_barrier()

        for g in range(G):
            def floop(s, carry, g=g):
                scat8(g * NCKM, s)
                return carry

            lax.fori_loop(0, NCK1 // 8, floop, 0)

    plsc.subcore_barrier()
    pltpu.sync_copy(acc.at[pl.ds(sid * DPT, DPT)],
                    out_hbm.at[cid, pl.ds(sid * DPT, DPT)])


_SC_CACHE = {}


def _deg_call(*args):
    if "deg" not in _SC_CACHE:
        _SC_CACHE["deg"] = pl.kernel(
            _deg_body,
            out_type=jax.ShapeDtypeStruct((NC, DEG_LEN), jnp.float32),
            mesh=plsc.VectorSubcoreMesh(core_axis_name="c",
                                        subcore_axis_name="s",
                                        num_cores=NC, num_subcores=NS),
            scratch_types=[
                pltpu.VMEM((G * NCKM, K), jnp.int32),
                pltpu.VMEM((G * NCKM, K), jnp.float32),
                pltpu.VMEM((DPT,), jnp.float32),
                pltpu.VMEM_SHARED((DEG_LEN,), jnp.float32),
                pltpu.SemaphoreType.DMA,
            ],
        )
    return _SC_CACHE["deg"](*args)


def _lane_bcast(vec, jj):
    idx = jnp.full((16, 1), jj, jnp.int32)
    return lax.gather(
        vec, idx,
        lax.GatherDimensionNumbers(offset_dims=(), collapsed_slice_dims=(0,),
                                   start_index_map=(0,)),
        (1,), mode=lax.GatherScatterMode.PROMISE_IN_BOUNDS)


def _seg_body(y_hbm, src_hbm, dst_hbm, ew_hbm, out_hbm,
              sbuf, dbuf, ebuf, rbuf, acc, isem0, isem1, gsem0, gsem1):
    cid = lax.axis_index("c")
    sid = lax.axis_index("s")
    wid = cid * NS + sid
    zero = jnp.zeros((16,), jnp.float32)
    isem = (isem0, isem1)
    gsem = (gsem0, gsem1)
    base = sid * RPT

    def start_idx(r, b):
        pltpu.async_copy(src_hbm.at[wid, r], sbuf.at[b], isem[b])
        pltpu.async_copy(dst_hbm.at[wid, r], dbuf.at[b], isem[b])
        pltpu.async_copy(ew_hbm.at[wid, r], ebuf.at[b], isem[b])

    def wait_idx(r, b):
        pltpu.make_async_copy(src_hbm.at[wid, r], sbuf.at[b], isem[b]).wait()
        pltpu.make_async_copy(dst_hbm.at[wid, r], dbuf.at[b], isem[b]).wait()
        pltpu.make_async_copy(ew_hbm.at[wid, r], ebuf.at[b], isem[b]).wait()

    def run_pipe(row0, nck):
        start_idx(row0 + 0, 0)
        start_idx(row0 + 1, 1)
        wait_idx(row0 + 0, 0)
        pltpu.async_copy(y_hbm.at[sbuf.at[0]], rbuf.at[0], gsem0)

        def pair(c2, carry):
            for b in range(2):
                c = 2 * c2 + b
                nb = 1 - b

                @pl.when(c < nck - 1)
                def _launch_next_gather():
                    wait_idx(row0 + c + 1, nb)
                    pltpu.async_copy(y_hbm.at[sbuf.at[nb]], rbuf.at[nb],
                                     gsem[nb])

                pltpu.make_async_copy(y_hbm.at[sbuf.at[b]], rbuf.at[b],
                                      gsem[b]).wait()

                def sgrp(q2, icarry):
                    ew16 = ebuf[b, pl.ds(q2 * 16, 16)]
                    for jj in range(16):
                        j = q2 * 16 + jj
                        w = _lane_bcast(ew16, jj)
                        for t in range(8):
                            sl = pl.ds(16 * t, 16)
                            rbuf[b, j, sl] = rbuf[b, j, sl] * w
                    return icarry

                lax.fori_loop(0, K // 16, sgrp, 0)
                pltpu.sync_copy(rbuf.at[b], acc.at[dbuf.at[b]], add=True)

                @pl.when(c < nck - 2)
                def _start_next_idx():
                    start_idx(row0 + c + 2, b)

            return carry

        lax.fori_loop(0, nck // 2, pair, 0)

    for g in range(G):
        row0 = g * NCKM

        def zrow(j, carry):
            for t in range(8):
                rbuf[0, j, pl.ds(16 * t, 16)] = zero
            return carry

        lax.fori_loop(0, K, zrow, 0)
        for b in range(RPT // K):
            pltpu.sync_copy(rbuf.at[0], acc.at[pl.ds(base + b * K, K)])
        plsc.subcore_barrier()

        @pl.when(cid == 0)
        def _pipe0():
            run_pipe(row0, NCK0)

        @pl.when(cid == 1)
        def _pipe1():
            run_pipe(row0, NCK1)

        plsc.subcore_barrier()
        for b in range(RPT // K):
            pltpu.sync_copy(acc.at[pl.ds(base + b * K, K)],
                            out_hbm.at[cid, g, pl.ds(base + b * K, K)])
        plsc.subcore_barrier()


def _seg_call(*args):
    if "seg" not in _SC_CACHE:
        _SC_CACHE["seg"] = pl.kernel(
            _seg_body,
            out_type=jax.ShapeDtypeStruct((NC, G, NPAD, D), jnp.float32),
            mesh=plsc.VectorSubcoreMesh(core_axis_name="c",
                                        subcore_axis_name="s",
                                        num_cores=NC, num_subcores=NS),
            scratch_types=[
                pltpu.VMEM((2, K), jnp.int32),
                pltpu.VMEM((2, K), jnp.int32),
                pltpu.VMEM((2, K), jnp.float32),
                pltpu.VMEM((2, K, D), jnp.float32),
                pltpu.VMEM_SHARED((NPAD, D), jnp.float32),
                pltpu.SemaphoreType.DMA,
                pltpu.SemaphoreType.DMA,
                pltpu.SemaphoreType.DMA,
                pltpu.SemaphoreType.DMA,
            ],
        )
    return _SC_CACHE["seg"](*args)


def _full(shape):
    return pl.BlockSpec(shape, lambda *a: tuple(0 for _ in shape))


def _dinv_body(d_ref, o_ref):
    o_ref[...] = lax.rsqrt(d_ref[0] + d_ref[1] + 1.0)


def _finish_deg(degacc):
    return pl.pallas_call(
        _dinv_body,
        out_shape=jax.ShapeDtypeStruct((G, NPAD), jnp.float32),
    )(degacc.reshape(NC, G, NPAD))


def _fc_body(x_ref, W1, b1, W2, b2, W3, b3, o_ref):
    h = jnp.dot(x_ref[...], W1[...], preferred_element_type=jnp.float32)
    h = jnp.maximum(h + b1[...], 0.0)
    h = jnp.dot(h, W2[...], preferred_element_type=jnp.float32)
    h = jnp.maximum(h + b2[...], 0.0)
    h = jnp.dot(h, W3[...], preferred_element_type=jnp.float32)
    o_ref[...] = jnp.maximum(h + b3[...], 0.0)


def _mlp(x, W1, b1, W2, b2, W3, b3):
    return pl.pallas_call(
        _fc_body,
        grid=(N // R,),
        in_specs=[pl.BlockSpec((R, D), lambda i: (i, 0)),
                  _full((D, 256)), _full((1, 256)),
                  _full((256, D)), _full((1, D)),
                  _full((D, 64)), _full((1, 64))],
        out_specs=pl.BlockSpec((R, 64), lambda i: (i, 0)),
        out_shape=jax.ShapeDtypeStruct((N, 64), jnp.float32),
    )(x, W1, b1.reshape(1, -1), W2, b2.reshape(1, -1), W3, b3.reshape(1, -1))


def _pre_body(x_ref, W_ref, dv_ref, y_ref):
    y_ref[0] = jnp.dot(x_ref[...], W_ref[0],
                       preferred_element_type=jnp.float32) * dv_ref[0]


def _pre3(x, W, dv):
    return pl.pallas_call(
        _pre_body,
        grid=(G, N // R),
        in_specs=[pl.BlockSpec((R, D), lambda g, i: (i, 0)),
                  pl.BlockSpec((1, D, D), lambda g, i: (g, 0, 0)),
                  pl.BlockSpec((1, R, 1), lambda g, i: (g, i, 0))],
        out_specs=pl.BlockSpec((1, R, D), lambda g, i: (g, i, 0)),
        out_shape=jax.ShapeDtypeStruct((G, NPAD, D), jnp.float32),
    )(x, W, dv)


def _mid_body(acc_ref, y_ref, dv_ref, b_ref, W_ref, o_ref):
    a = acc_ref[0, 0] + acc_ref[1, 0] + y_ref[0]
    h = jnp.maximum(a * dv_ref[0] + b_ref[0], 0.0)
    o_ref[0] = jnp.dot(h, W_ref[0],
                       preferred_element_type=jnp.float32) * dv_ref[0]


def _mid3(acc, y, dv, b, W):
    return pl.pallas_call(
        _mid_body,
        grid=(G, N // R),
        in_specs=[pl.BlockSpec((NC, 1, R, D), lambda g, i: (0, g, i, 0)),
                  pl.BlockSpec((1, R, D), lambda g, i: (g, i, 0)),
                  pl.BlockSpec((1, R, 1), lambda g, i: (g, i, 0)),
                  pl.BlockSpec((1, 1, D), lambda g, i: (g, 0, 0)),
                  pl.BlockSpec((1, D, D), lambda g, i: (g, 0, 0))],
        out_specs=pl.BlockSpec((1, R, D), lambda g, i: (g, i, 0)),
        out_shape=jax.ShapeDtypeStruct((G, NPAD, D), jnp.float32),
    )(acc, y, dv, b, W)


def _post_body(acc_ref, y_ref, dv_ref, b_ref, W1, b1, W2, b2, W3, b3, o_ref):
    a = acc_ref[0, 0] + acc_ref[1, 0] + y_ref[0]
    h = jnp.maximum(a * dv_ref[0] + b_ref[0], 0.0)
    h = jnp.dot(h, W1[...], preferred_element_type=jnp.float32)
    h = jnp.maximum(h + b1[...], 0.0)
    h = jnp.dot(h, W2[...], preferred_element_type=jnp.float32)
    h = jnp.maximum(h + b2[...], 0.0)
    h = jnp.dot(h, W3[...], preferred_element_type=jnp.float32)
    o_ref[0] = jnp.maximum(h + b3[...], 0.0)


def _post3(acc, y, dv, b, W1, b1, W2, b2, W3, b3):
    return pl.pallas_call(
        _post_body,
        grid=(G, N // R),
        in_specs=[pl.BlockSpec((NC, 1, R, D), lambda g, i: (0, g, i, 0)),
                  pl.BlockSpec((1, R, D), lambda g, i: (g, i, 0)),
                  pl.BlockSpec((1, R, 1), lambda g, i: (g, i, 0)),
                  pl.BlockSpec((1, 1, D), lambda g, i: (g, 0, 0)),
                  _full((D, 256)), _full((1, 256)),
                  _full((256, D)), _full((1, D)),
                  _full((D, 64)), _full((1, 64))],
        out_specs=pl.BlockSpec((1, R, 64), lambda g, i: (g, i, 0)),
        out_shape=jax.ShapeDtypeStruct((G, N, 64), jnp.float32),
    )(acc, y, dv, b, W1, b1.reshape(1, -1),
      W2, b2.reshape(1, -1), W3, b3.reshape(1, -1))


def _addf_body(x0_ref, o_ref, out_ref):
    out_ref[...] = x0_ref[...] + o_ref[0] + o_ref[1] + o_ref[2]


def _addf(x0, o):
    return pl.pallas_call(
        _addf_body,
        grid=(N // R,),
        in_specs=[pl.BlockSpec((R, 64), lambda i: (i, 0)),
                  pl.BlockSpec((G, R, 64), lambda i: (0, i, 0))],
        out_specs=pl.BlockSpec((R, 64), lambda i: (i, 0)),
        out_shape=jax.ShapeDtypeStruct((N, 64), jnp.float32),
    )(x0, o)


def kernel(X, edge_index1, edge_weight1, edge_index2, edge_weight2,
           edge_index3, edge_weight3,
           gcn1_W1, gcn1_b1, gcn2_W1, gcn2_b1,
           gcn1_W2, gcn1_b2, gcn2_W2, gcn2_b2,
           gcn1_W3, gcn1_b3, gcn2_W3, gcn2_b3,
           lin1_W1, lin1_b1, lin1_W2, lin1_b2, lin1_W3, lin1_b3,
           lin2_W1, lin2_b1, lin2_W2, lin2_b2, lin2_W3, lin2_b3):
    srcs, dsts, doffs, ews = [], [], [], []
    for g, (ei, ew) in enumerate([(edge_index1, edge_weight1),
                                  (edge_index2, edge_weight2),
                                  (edge_index3, edge_weight3)]):
        srcoff, dst, dstoff, eww = _prep_edges(ei, ew, g * NPAD)
        srcs.append(srcoff)
        dsts.append(dst)
        doffs.append(dstoff)
        ews.append(eww)
    src_all = jnp.concatenate(srcs, axis=1)
    dst_all = jnp.concatenate(dsts, axis=1)
    doff_all = jnp.concatenate(doffs, axis=1)
    ew_all = jnp.concatenate(ews, axis=1)

    degacc = _deg_call(doff_all, ew_all)
    dinv_all = _finish_deg(degacc)
    dv = dinv_all[:, :N].reshape(G, N, 1)

    X0 = _mlp(X, lin1_W1, lin1_b1, lin1_W2, lin1_b2, lin1_W3, lin1_b3)

    Wa = jnp.stack([gcn1_W1, gcn1_W2, gcn1_W3])
    ba = jnp.stack([gcn1_b1, gcn1_b2, gcn1_b3]).reshape(G, 1, D)
    Wb = jnp.stack([gcn2_W1, gcn2_W2, gcn2_W3])
    bb = jnp.stack([gcn2_b1, gcn2_b2, gcn2_b3]).reshape(G, 1, D)

    y1 = _pre3(X, Wa, dv)
    acc1 = _seg_call(y1.reshape(G * NPAD, D), src_all, dst_all, ew_all)
    y2 = _mid3(acc1, y1, dv, ba, Wb)
    acc2 = _seg_call(y2.reshape(G * NPAD, D), src_all, dst_all, ew_all)
    o = _post3(acc2, y2, dv, bb,
               lin2_W1, lin2_b1, lin2_W2, lin2_b2, lin2_W3, lin2_b3)

    Xout = _addf(X0, o)
    return (Xout, o[0], o[1], o[2])


# speedup vs baseline: 1.0756x; 1.0354x over previous
import jax
import jax.numpy as jnp
from jax import lax
from jax.experimental import pallas as pl
from jax.experimental.pallas import tpu as pltpu
from jax.experimental.pallas import tpu_sc as plsc

N = 10000
E = 320000
D = 128
G = 3
NC = 2
NS = 16
NW = NC * NS
K = 128
NCKG = 160
ROWS = 2 * NCKG
NPAD = 10240
RPT = NPAD // NS
DEG_LEN = G * NPAD
DPT = DEG_LEN // NS
R = 1000


def _prep_graph(edge_index, edge_weight, goff):
    pad = NS * NCKG * K - E

    def shp(a):
        return a.reshape(NS, NCKG, K)

    zi = jnp.zeros((pad,), jnp.int32)
    src = shp(jnp.concatenate([edge_index[0], zi]) + goff)
    dstf = jnp.concatenate([edge_index[1], zi])
    ew = shp(jnp.concatenate([edge_weight, jnp.zeros((pad,), jnp.float32)]))
    return src, shp(dstf), shp(dstf + goff), ew


def _deg_body(dst_hbm, ew_hbm, out_hbm, dstv, ewv, zb, acc, sem):
    cid = lax.axis_index("c")
    sid = lax.axis_index("s")
    wid = cid * NS + sid
    zero = jnp.zeros((16,), jnp.float32)

    def zloop(i, carry):
        zb[pl.ds(pl.multiple_of(i * 16, 16), 16)] = zero
        return carry

    lax.fori_loop(0, DPT // 16, zloop, 0)
    pltpu.sync_copy(zb, acc.at[pl.ds(sid * DPT, DPT)])

    def scat8(s, carry):
        for q in range(8):
            c = s * 8 + q
            pltpu.async_copy(ewv.at[c], acc.at[dstv.at[c]], sem, add=True)
        for q in range(8):
            c = s * 8 + q
            pltpu.make_async_copy(ewv.at[c], acc.at[dstv.at[c]], sem).wait()
        return carry

    @pl.when(cid == 0)
    def _core0():
        pltpu.sync_copy(dst_hbm.at[wid], dstv)
        pltpu.sync_copy(ew_hbm.at[wid], ewv)
        plsc.subcore_barrier()
        lax.fori_loop(0, ROWS // 8, scat8, 0)

    @pl.when(cid == 1)
    def _core1():
        sl = pl.ds(0, NCKG)
        pltpu.sync_copy(dst_hbm.at[wid, sl], dstv.at[sl])
        pltpu.sync_copy(ew_hbm.at[wid, sl], ewv.at[sl])
        plsc.subcore_barrier()
        lax.fori_loop(0, NCKG // 8, scat8, 0)

    plsc.subcore_barrier()
    pltpu.sync_copy(acc.at[pl.ds(sid * DPT, DPT)],
                    out_hbm.at[cid, pl.ds(sid * DPT, DPT)])


_SC_CACHE = {}


def _deg_call(*args):
    if "deg" not in _SC_CACHE:
        _SC_CACHE["deg"] = pl.kernel(
            _deg_body,
            out_type=jax.ShapeDtypeStruct((NC, DEG_LEN), jnp.float32),
            mesh=plsc.VectorSubcoreMesh(core_axis_name="c",
                                        subcore_axis_name="s",
                                        num_cores=NC, num_subcores=NS),
            scratch_types=[
                pltpu.VMEM((ROWS, K), jnp.int32),
                pltpu.VMEM((ROWS, K), jnp.float32),
                pltpu.VMEM((DPT,), jnp.float32),
                pltpu.VMEM_SHARED((DEG_LEN,), jnp.float32),
                pltpu.SemaphoreType.DMA,
            ],
        )
    return _SC_CACHE["deg"](*args)


def _lane_bcast(vec, jj):
    idx = jnp.full((16, 1), jj, jnp.int32)
    return lax.gather(
        vec, idx,
        lax.GatherDimensionNumbers(offset_dims=(), collapsed_slice_dims=(0,),
                                   start_index_map=(0,)),
        (1,), mode=lax.GatherScatterMode.PROMISE_IN_BOUNDS)


def _seg_body(y_hbm, src_hbm, dst_hbm, ew_hbm, out_hbm,
              sbuf, dbuf, ebuf, rbuf, acc, isem0, isem1, gsem0, gsem1):
    cid = lax.axis_index("c")
    sid = lax.axis_index("s")
    wid = cid * NS + sid
    zero = jnp.zeros((16,), jnp.float32)
    isem = (isem0, isem1)
    gsem = (gsem0, gsem1)
    base = sid * RPT

    def start_idx(r, b):
        pltpu.async_copy(src_hbm.at[wid, r], sbuf.at[b], isem[b])
        pltpu.async_copy(dst_hbm.at[wid, r], dbuf.at[b], isem[b])
        pltpu.async_copy(ew_hbm.at[wid, r], ebuf.at[b], isem[b])

    def wait_idx(r, b):
        pltpu.make_async_copy(src_hbm.at[wid, r], sbuf.at[b], isem[b]).wait()
        pltpu.make_async_copy(dst_hbm.at[wid, r], dbuf.at[b], isem[b]).wait()
        pltpu.make_async_copy(ew_hbm.at[wid, r], ebuf.at[b], isem[b]).wait()

    def run_pipe(row0):
        start_idx(row0 + 0, 0)
        start_idx(row0 + 1, 1)
        wait_idx(row0 + 0, 0)
        pltpu.async_copy(y_hbm.at[sbuf.at[0]], rbuf.at[0], gsem0)

        def pair(c2, carry):
            for b in range(2):
                c = 2 * c2 + b
                nb = 1 - b

                @pl.when(c < NCKG - 1)
                def _launch_next_gather():
                    wait_idx(row0 + c + 1, nb)
                    pltpu.async_copy(y_hbm.at[sbuf.at[nb]], rbuf.at[nb],
                                     gsem[nb])

                pltpu.make_async_copy(y_hbm.at[sbuf.at[b]], rbuf.at[b],
                                      gsem[b]).wait()

                def sgrp(q2, icarry):
                    ew16 = ebuf[b, pl.ds(q2 * 16, 16)]
                    for jj in range(16):
                        j = q2 * 16 + jj
                        w = _lane_bcast(ew16, jj)
                        for t in range(8):
                            sl = pl.ds(16 * t, 16)
                            rbuf[b, j, sl] = rbuf[b, j, sl] * w
                    return icarry

                lax.fori_loop(0, K // 16, sgrp, 0)
                pltpu.sync_copy(rbuf.at[b], acc.at[dbuf.at[b]], add=True)

                @pl.when(c < NCKG - 2)
                def _start_next_idx():
                    start_idx(row0 + c + 2, b)

            return carry

        lax.fori_loop(0, NCKG // 2, pair, 0)

    def do_graph(g, row0):
        def zrow(j, carry):
            for t in range(8):
                rbuf[0, j, pl.ds(16 * t, 16)] = zero
            return carry

        lax.fori_loop(0, K, zrow, 0)
        for b in range(RPT // K):
            pltpu.sync_copy(rbuf.at[0], acc.at[pl.ds(base + b * K, K)])
        plsc.subcore_barrier()
        run_pipe(row0)
        plsc.subcore_barrier()
        for b in range(RPT // K):
            pltpu.sync_copy(acc.at[pl.ds(base + b * K, K)],
                            out_hbm.at[g, pl.ds(base + b * K, K)])

    @pl.when(cid == 0)
    def _core0():
        do_graph(0, 0)
        do_graph(1, NCKG)

    @pl.when(cid == 1)
    def _core1():
        do_graph(2, 0)

    plsc.subcore_barrier()


def _seg_call(*args):
    if "seg" not in _SC_CACHE:
        _SC_CACHE["seg"] = pl.kernel(
            _seg_body,
            out_type=jax.ShapeDtypeStruct((G, NPAD, D), jnp.float32),
            mesh=plsc.VectorSubcoreMesh(core_axis_name="c",
                                        subcore_axis_name="s",
                                        num_cores=NC, num_subcores=NS),
            scratch_types=[
                pltpu.VMEM((2, K), jnp.int32),
                pltpu.VMEM((2, K), jnp.int32),
                pltpu.VMEM((2, K), jnp.float32),
                pltpu.VMEM((2, K, D), jnp.float32),
                pltpu.VMEM_SHARED((NPAD, D), jnp.float32),
                pltpu.SemaphoreType.DMA,
                pltpu.SemaphoreType.DMA,
                pltpu.SemaphoreType.DMA,
                pltpu.SemaphoreType.DMA,
            ],
        )
    return _SC_CACHE["seg"](*args)


def _full(shape):
    return pl.BlockSpec(shape, lambda *a: tuple(0 for _ in shape))


def _dinv_body(d_ref, o_ref):
    o_ref[...] = lax.rsqrt(d_ref[0] + d_ref[1] + 1.0)


def _finish_deg(degacc):
    return pl.pallas_call(
        _dinv_body,
        out_shape=jax.ShapeDtypeStruct((G, NPAD), jnp.float32),
    )(degacc.reshape(NC, G, NPAD))


def _fc_body(x_ref, W1, b1, W2, b2, W3, b3, o_ref):
    h = jnp.dot(x_ref[...], W1[...], preferred_element_type=jnp.float32)
    h = jnp.maximum(h + b1[...], 0.0)
    h = jnp.dot(h, W2[...], preferred_element_type=jnp.float32)
    h = jnp.maximum(h + b2[...], 0.0)
    h = jnp.dot(h, W3[...], preferred_element_type=jnp.float32)
    o_ref[...] = jnp.maximum(h + b3[...], 0.0)


def _mlp(x, W1, b1, W2, b2, W3, b3):
    return pl.pallas_call(
        _fc_body,
        grid=(N // R,),
        in_specs=[pl.BlockSpec((R, D), lambda i: (i, 0)),
                  _full((D, 256)), _full((1, 256)),
                  _full((256, D)), _full((1, D)),
                  _full((D, 64)), _full((1, 64))],
        out_specs=pl.BlockSpec((R, 64), lambda i: (i, 0)),
        out_shape=jax.ShapeDtypeStruct((N, 64), jnp.float32),
    )(x, W1, b1.reshape(1, -1), W2, b2.reshape(1, -1), W3, b3.reshape(1, -1))


def _pre_body(x_ref, W_ref, dv_ref, y_ref):
    y_ref[0] = jnp.dot(x_ref[...], W_ref[0],
                       preferred_element_type=jnp.float32) * dv_ref[0]


def _pre3(x, W, dv):
    return pl.pallas_call(
        _pre_body,
        grid=(G, N // R),
        in_specs=[pl.BlockSpec((R, D), lambda g, i: (i, 0)),
                  pl.BlockSpec((1, D, D), lambda g, i: (g, 0, 0)),
                  pl.BlockSpec((1, R, 1), lambda g, i: (g, i, 0))],
        out_specs=pl.BlockSpec((1, R, D), lambda g, i: (g, i, 0)),
        out_shape=jax.ShapeDtypeStruct((G, NPAD, D), jnp.float32),
    )(x, W, dv)


def _mid_body(acc_ref, y_ref, dv_ref, b_ref, W_ref, o_ref):
    a = acc_ref[0] + y_ref[0]
    h = jnp.maximum(a * dv_ref[0] + b_ref[0], 0.0)
    o_ref[0] = jnp.dot(h, W_ref[0],
                       preferred_element_type=jnp.float32) * dv_ref[0]


def _mid3(acc, y, dv, b, W):
    return pl.pallas_call(
        _mid_body,
        grid=(G, N // R),
        in_specs=[pl.BlockSpec((1, R, D), lambda g, i: (g, i, 0)),
                  pl.BlockSpec((1, R, D), lambda g, i: (g, i, 0)),
                  pl.BlockSpec((1, R, 1), lambda g, i: (g, i, 0)),
                  pl.BlockSpec((1, 1, D), lambda g, i: (g, 0, 0)),
                  pl.BlockSpec((1, D, D), lambda g, i: (g, 0, 0))],
        out_specs=pl.BlockSpec((1, R, D), lambda g, i: (g, i, 0)),
        out_shape=jax.ShapeDtypeStruct((G, NPAD, D), jnp.float32),
    )(acc, y, dv, b, W)


def _post_body(acc_ref, y_ref, dv_ref, b_ref, W1, b1, W2, b2, W3, b3, o_ref):
    a = acc_ref[0] + y_ref[0]
    h = jnp.maximum(a * dv_ref[0] + b_ref[0], 0.0)
    h = jnp.dot(h, W1[...], preferred_element_type=jnp.float32)
    h = jnp.maximum(h + b1[...], 0.0)
    h = jnp.dot(h, W2[...], preferred_element_type=jnp.float32)
    h = jnp.maximum(h + b2[...], 0.0)
    h = jnp.dot(h, W3[...], preferred_element_type=jnp.float32)
    o_ref[0] = jnp.maximum(h + b3[...], 0.0)


def _post3(acc, y, dv, b, W1, b1, W2, b2, W3, b3):
    return pl.pallas_call(
        _post_body,
        grid=(G, N // R),
        in_specs=[pl.BlockSpec((1, R, D), lambda g, i: (g, i, 0)),
                  pl.BlockSpec((1, R, D), lambda g, i: (g, i, 0)),
                  pl.BlockSpec((1, R, 1), lambda g, i: (g, i, 0)),
                  pl.BlockSpec((1, 1, D), lambda g, i: (g, 0, 0)),
                  _full((D, 256)), _full((1, 256)),
                  _full((256, D)), _full((1, D)),
                  _full((D, 64)), _full((1, 64))],
        out_specs=pl.BlockSpec((1, R, 64), lambda g, i: (g, i, 0)),
        out_shape=jax.ShapeDtypeStruct((G, N, 64), jnp.float32),
    )(acc, y, dv, b, W1, b1.reshape(1, -1),
      W2, b2.reshape(1, -1), W3, b3.reshape(1, -1))


def _addf_body(x0_ref, o_ref, out_ref):
    out_ref[...] = x0_ref[...] + o_ref[0] + o_ref[1] + o_ref[2]


def _addf(x0, o):
    return pl.pallas_call(
        _addf_body,
        grid=(N // R,),
        in_specs=[pl.BlockSpec((R, 64), lambda i: (i, 0)),
                  pl.BlockSpec((G, R, 64), lambda i: (0, i, 0))],
        out_specs=pl.BlockSpec((R, 64), lambda i: (i, 0)),
        out_shape=jax.ShapeDtypeStruct((N, 64), jnp.float32),
    )(x0, o)


def kernel(X, edge_index1, edge_weight1, edge_index2, edge_weight2,
           edge_index3, edge_weight3,
           gcn1_W1, gcn1_b1, gcn2_W1, gcn2_b1,
           gcn1_W2, gcn1_b2, gcn2_W2, gcn2_b2,
           gcn1_W3, gcn1_b3, gcn2_W3, gcn2_b3,
           lin1_W1, lin1_b1, lin1_W2, lin1_b2, lin1_W3, lin1_b3,
           lin2_W1, lin2_b1, lin2_W2, lin2_b2, lin2_W3, lin2_b3):
    p0 = _prep_graph(edge_index1, edge_weight1, 0)
    p1 = _prep_graph(edge_index2, edge_weight2, NPAD)
    p2 = _prep_graph(edge_index3, edge_weight3, 2 * NPAD)

    def asm(i, dtype):
        c0 = jnp.concatenate([p0[i], p1[i]], axis=1)
        c1 = jnp.concatenate(
            [p2[i], jnp.zeros((NS, NCKG, K), dtype)], axis=1)
        return jnp.concatenate([c0, c1], axis=0)

    src_all = asm(0, jnp.int32)
    dst_all = asm(1, jnp.int32)
    doff_all = asm(2, jnp.int32)
    ew_all = asm(3, jnp.float32)

    degacc = _deg_call(doff_all, ew_all)
    dinv_all = _finish_deg(degacc)
    dv = dinv_all[:, :N].reshape(G, N, 1)

    X0 = _mlp(X, lin1_W1, lin1_b1, lin1_W2, lin1_b2, lin1_W3, lin1_b3)

    Wa = jnp.stack([gcn1_W1, gcn1_W2, gcn1_W3])
    ba = jnp.stack([gcn1_b1, gcn1_b2, gcn1_b3]).reshape(G, 1, D)
    Wb = jnp.stack([gcn2_W1, gcn2_W2, gcn2_W3])
    bb = jnp.stack([gcn2_b1, gcn2_b2, gcn2_b3]).reshape(G, 1, D)

    y1 = _pre3(X, Wa, dv)
    acc1 = _seg_call(y1.reshape(G * NPAD, D), src_all, dst_all, ew_all)
    y2 = _mid3(acc1, y1, dv, ba, Wb)
    acc2 = _seg_call(y2.reshape(G * NPAD, D), src_all, dst_all, ew_all)
    o = _post3(acc2, y2, dv, bb,
               lin2_W1, lin2_b1, lin2_W2, lin2_b2, lin2_W3, lin2_b3)

    Xout = _addf(X0, o)
    return (Xout, o[0], o[1], o[2])
